# baseline (speedup 1.0000x reference)
import functools

import jax
import jax.numpy as jnp
from jax.experimental import pallas as pl
from jax.experimental.pallas import tpu as pltpu


def _pick_tile(n, candidates):
    for t in candidates:
        if n % t == 0:
            return t
    return 128


def _fused_kernel(adj_ref, x0_ref, feats_ref, w_ref, b_ref,
                  items_ref, v_ref, t_ref, x1_ref, *, tm, emb):
    l = pl.program_id(0)
    i = pl.program_id(1)

    proj = jnp.dot(feats_ref[...], w_ref[...],
                   preferred_element_type=jnp.float32) + b_ref[...]
    v_ref[...] = proj[:, :emb]
    t_ref[...] = proj[:, emb:]

    @pl.when(l == 0)
    def _():
        x1_ref[pl.ds(i * tm, tm), :] = jnp.dot(
            adj_ref[...], x0_ref[...], preferred_element_type=jnp.float32)

    @pl.when(l == 1)
    def _():
        items_ref[...] = jnp.dot(adj_ref[...], x1_ref[...],
                                 preferred_element_type=jnp.float32)


def kernel(mAdj, itemEmbds, featsPadded, wBlk, bCat):
    n, emb = itemEmbds.shape
    k_pad = featsPadded.shape[1]
    out_w = wBlk.shape[1]

    tm = _pick_tile(n, (1024, 512, 256, 128))
    n_i = n // tm
    tf = tm // 2

    flops = 2 * (2 * n * n * emb + n * k_pad * out_w)
    bytes_accessed = 4 * (2 * n * n + n * k_pad + n * emb
                          + k_pad * out_w + out_w + 3 * n * emb)

    items, v, t = pl.pallas_call(
        functools.partial(_fused_kernel, tm=tm, emb=emb),
        out_shape=[jax.ShapeDtypeStruct((n, emb), jnp.float32),
                   jax.ShapeDtypeStruct((n, emb), jnp.float32),
                   jax.ShapeDtypeStruct((n, emb), jnp.float32)],
        grid_spec=pltpu.PrefetchScalarGridSpec(
            num_scalar_prefetch=0,
            grid=(2, n_i),
            in_specs=[
                pl.BlockSpec((tm, n), lambda l, i: (i, 0)),
                pl.BlockSpec((n, emb), lambda l, i: (0, 0)),
                pl.BlockSpec((tf, k_pad), lambda l, i: (l * n_i + i, 0)),
                pl.BlockSpec((k_pad, out_w), lambda l, i: (0, 0)),
                pl.BlockSpec((1, out_w), lambda l, i: (0, 0)),
            ],
            out_specs=[
                pl.BlockSpec((tm, emb),
                             lambda l, i: (jnp.where(l == 1, i, 0), 0)),
                pl.BlockSpec((tf, emb), lambda l, i: (l * n_i + i, 0)),
                pl.BlockSpec((tf, emb), lambda l, i: (l * n_i + i, 0)),
            ],
            scratch_shapes=[pltpu.VMEM((n, emb), jnp.float32)]),
        compiler_params=pltpu.CompilerParams(
            dimension_semantics=("arbitrary", "arbitrary")),
        cost_estimate=pl.CostEstimate(flops=flops, transcendentals=0,
                                      bytes_accessed=bytes_accessed),
    )(mAdj, itemEmbds, featsPadded, wBlk, bCat)

    return items, v, t


# speedup vs baseline: 1.1660x; 1.1660x over previous
import functools

import jax
import jax.numpy as jnp
from jax.experimental import pallas as pl
from jax.experimental.pallas import tpu as pltpu


def _pick_tile(n, candidates):
    for t in candidates:
        if n % t == 0:
            return t
    return 128


def _fused_kernel(adj_ref, x0_ref, feats_ref, w_ref, b_ref,
                  items_ref, v_ref, t_ref, a16_ref, x1_ref, x1c_ref,
                  *, tm, emb):
    l = pl.program_id(0)
    i = pl.program_id(1)

    @pl.when(l == 0)
    def _():
        adj = adj_ref[...]
        a16_ref[pl.ds(i * tm, tm), :] = adj.astype(jnp.bfloat16)
        x1_ref[pl.ds(i * tm, tm), :] = jnp.dot(
            adj, x0_ref[...], preferred_element_type=jnp.float32)
        proj = jnp.dot(feats_ref[...], w_ref[...],
                       preferred_element_type=jnp.float32) + b_ref[...]
        v_ref[...] = proj[:, :emb]
        t_ref[...] = proj[:, emb:]

    @pl.when(l == 1)
    def _():
        @pl.when(i == 0)
        def _():
            x1c_ref[...] = x1_ref[...].astype(jnp.bfloat16)

        items_ref[...] = jnp.dot(a16_ref[pl.ds(i * tm, tm), :],
                                 x1c_ref[...],
                                 preferred_element_type=jnp.float32)


def kernel(mAdj, itemEmbds, featsPadded, wBlk, bCat):
    n, emb = itemEmbds.shape
    k_pad = featsPadded.shape[1]
    out_w = wBlk.shape[1]

    tm = _pick_tile(n, (256, 128))
    n_i = n // tm
    last = n_i - 1

    flops = 2 * (2 * n * n * emb + n * k_pad * out_w)
    bytes_accessed = 4 * (n * n + n * k_pad + n * emb
                          + k_pad * out_w + out_w + 3 * n * emb)

    items, v, t = pl.pallas_call(
        functools.partial(_fused_kernel, tm=tm, emb=emb),
        out_shape=[jax.ShapeDtypeStruct((n, emb), jnp.float32),
                   jax.ShapeDtypeStruct((n, emb), jnp.float32),
                   jax.ShapeDtypeStruct((n, emb), jnp.float32)],
        grid_spec=pltpu.PrefetchScalarGridSpec(
            num_scalar_prefetch=0,
            grid=(2, n_i),
            in_specs=[
                pl.BlockSpec((tm, n),
                             lambda l, i: (jnp.where(l == 0, i, last), 0)),
                pl.BlockSpec((n, emb), lambda l, i: (0, 0)),
                pl.BlockSpec((tm, k_pad),
                             lambda l, i: (jnp.where(l == 0, i, last), 0)),
                pl.BlockSpec((k_pad, out_w), lambda l, i: (0, 0)),
                pl.BlockSpec((1, out_w), lambda l, i: (0, 0)),
            ],
            out_specs=[
                pl.BlockSpec((tm, emb),
                             lambda l, i: (jnp.where(l == 1, i, 0), 0)),
                pl.BlockSpec((tm, emb),
                             lambda l, i: (jnp.where(l == 0, i, last), 0)),
                pl.BlockSpec((tm, emb),
                             lambda l, i: (jnp.where(l == 0, i, last), 0)),
            ],
            scratch_shapes=[pltpu.VMEM((n, n), jnp.bfloat16),
                            pltpu.VMEM((n, emb), jnp.float32),
                            pltpu.VMEM((n, emb), jnp.bfloat16)]),
        compiler_params=pltpu.CompilerParams(
            dimension_semantics=("arbitrary", "arbitrary")),
        cost_estimate=pl.CostEstimate(flops=flops, transcendentals=0,
                                      bytes_accessed=bytes_accessed),
    )(mAdj, itemEmbds, featsPadded, wBlk, bCat)

    return items, v, t
